# baseline (speedup 1.0000x reference)
import functools

import jax
import jax.numpy as jnp
from jax import lax
from jax.experimental import pallas as pl
from jax.experimental.pallas import tpu as pltpu
from jax.experimental.pallas import tpu_sc as plsc

L1 = 512
L2 = 16
L3 = 32
NUM_PSQT = 8
NUM_LS = 8
D = L1 + NUM_PSQT
DT = 544
DP = DT
KA = 32
NC = 2
NS = 16
NW = NC * NS


def _embedding_bag_sc(table16, idx):
    nbags = idx.shape[0]
    bags_per_w = nbags // NW
    mesh = plsc.VectorSubcoreMesh(
        core_axis_name="c", subcore_axis_name="s",
        num_cores=NC, num_subcores=NS)

    nbuf = 4
    nchunk = DT // 32

    @functools.partial(
        pl.kernel,
        out_type=jax.ShapeDtypeStruct((nbags, DP), jnp.float32),
        mesh=mesh,
        compiler_params=pltpu.CompilerParams(use_tc_tiling_on_sc=False,
                                             needs_layout_passes=False),
        scratch_types=[
            pltpu.VMEM((bags_per_w, KA), jnp.int32),
            [pltpu.VMEM((KA, DT), jnp.bfloat16) for _ in range(nbuf)],
            [pltpu.VMEM((DP,), jnp.float32) for _ in range(2)],
            [pltpu.SemaphoreType.DMA for _ in range(nbuf)],
            [pltpu.SemaphoreType.DMA for _ in range(2)],
        ],
    )
    def k(table_hbm, idx_hbm, out_hbm, idx_v, bufs, stages, sems, osems):
        wid = lax.axis_index("s") * NC + lax.axis_index("c")
        base = wid * bags_per_w
        pltpu.sync_copy(idx_hbm.at[pl.ds(base, bags_per_w)], idx_v)

        iota16 = lax.iota(jnp.int32, 16)
        even_i = iota16 * 2
        odd_i = even_i + 1
        zero16 = jnp.zeros((16,), jnp.float32)

        for b in range(nbuf - 1):
            pltpu.async_copy(table_hbm.at[idx_v.at[b]],
                             bufs[b], sems[b])

        def group_body(p, carry):
            for b in range(nbuf):
                g = p * nbuf + b
                par = b % 2
                stage, osem = stages[par], osems[par]
                gn = g + nbuf - 1
                nb = (b + nbuf - 1) % nbuf

                @pl.when(gn < bags_per_w)
                def _issue():
                    pltpu.async_copy(table_hbm.at[idx_v.at[gn]],
                                     bufs[nb], sems[nb])

                pltpu.make_async_copy(
                    table_hbm.at[idx_v.at[g]], bufs[b], sems[b]).wait()

                @pl.when(g >= 2)
                def _wait_out():
                    pltpu.make_async_copy(
                        stage, out_hbm.at[base], osem).wait()

                def row_body(j, accs):
                    new = []
                    for c in range(nchunk):
                        v = bufs[b][j, pl.ds(c * 32, 32)]
                        lo, hi = plsc.unpack(v, format=plsc.PackFormat.INTERLEAVED)
                        new.append(accs[2 * c] + lo)
                        new.append(accs[2 * c + 1] + hi)
                    return tuple(new)

                accs = lax.fori_loop(0, KA, row_body,
                                     tuple(zero16 for _ in range(2 * nchunk)))
                for c in range(nchunk):
                    off = jnp.int32(32 * c)
                    plsc.store_scatter(stage, [even_i + off], accs[2 * c])
                    plsc.store_scatter(stage, [odd_i + off], accs[2 * c + 1])
                pltpu.async_copy(stage, out_hbm.at[base + g], osem)
            return carry

        lax.fori_loop(0, bags_per_w // nbuf, group_body, 0)
        for par in range(2):
            pltpu.make_async_copy(stages[par], out_hbm.at[base],
                                  osems[par]).wait()

    return k(table16, idx)


def _prep_table_tc(feat_w):
    nrows = feat_w.shape[0]
    blk = 512
    def body(in_ref, out_ref):
        x = in_ref[...]
        z = jnp.zeros((blk, DT - D), jnp.float32)
        out_ref[...] = jnp.concatenate([x, z], axis=1).astype(jnp.bfloat16)
    return pl.pallas_call(
        body,
        grid=(nrows // blk,),
        in_specs=[pl.BlockSpec((blk, D), lambda i: (i, 0))],
        out_specs=pl.BlockSpec((blk, DT), lambda i: (i, 0)),
        out_shape=jax.ShapeDtypeStruct((nrows, DT), jnp.bfloat16),
    )(feat_w)


def _dense_tc(bags, us, pidx, lsidx, fb_main, w1t, b1r, w2t, b2r, w3t, b3r):
    batch = us.shape[0]
    blk = 512
    nblk = batch // blk
    scale = 127.0 / 128.0

    def body(wp_ref, bp_ref, us_ref, pidx_ref, ls_ref, fb_ref,
             w1_ref, b1_ref, w2_ref, b2_ref, w3_ref, b3_ref, out_ref):
        wp = wp_ref[...]
        bp = bp_ref[...]
        usv = us_ref[...]
        fb = fb_ref[...]
        w = jnp.clip(wp[:, :L1] + fb, 0.0, 1.0)
        b = jnp.clip(bp[:, :L1] + fb, 0.0, 1.0)
        half = L1 // 2
        w_pw = w[:, :half] * w[:, half:L1]
        b_pw = b[:, :half] * b[:, half:L1]
        usb = usv > 0.5
        l0 = jnp.concatenate(
            [jnp.where(usb, w_pw, b_pw), jnp.where(usb, b_pw, w_pw)],
            axis=1) * scale

        pidxv = pidx_ref[...]
        poh = (pidxv == lax.broadcasted_iota(jnp.int32, (blk, NUM_PSQT), 1)
               ).astype(jnp.float32)
        dps = wp[:, L1:D] - bp[:, L1:D]
        psqt = jnp.sum(dps * poh, axis=1, keepdims=True) * (usv - 0.5)

        lsv = ls_ref[...]
        l1x_all = jnp.dot(l0, w1_ref[...],
                          preferred_element_type=jnp.float32) + b1_ref[...]
        nsel = L2 + 1
        sel1 = jnp.zeros((blk, nsel), jnp.float32)
        for kk in range(NUM_LS):
            sel1 = sel1 + jnp.where(lsv == kk,
                                    l1x_all[:, kk * nsel:(kk + 1) * nsel], 0.0)
        l1x_ = sel1[:, :L2]
        l1x_out = sel1[:, L2:L2 + 1]
        l1c = jnp.clip(l1x_, 0.0, 1.0)
        l1cat = jnp.concatenate([l1c * l1c, l1c], axis=1) * scale

        l2x_all = jnp.dot(l1cat, w2_ref[...],
                          preferred_element_type=jnp.float32) + b2_ref[...]
        sel2 = jnp.zeros((blk, L3), jnp.float32)
        for kk in range(NUM_LS):
            sel2 = sel2 + jnp.where(lsv == kk,
                                    l2x_all[:, kk * L3:(kk + 1) * L3], 0.0)
        l2x = jnp.clip(sel2, 0.0, 1.0)

        l3_all = jnp.dot(l2x, w3_ref[...],
                         preferred_element_type=jnp.float32) + b3_ref[...]
        lsoh = (lsv == lax.broadcasted_iota(jnp.int32, (blk, NUM_LS), 1)
                ).astype(jnp.float32)
        l3sel = jnp.sum(l3_all * lsoh, axis=1, keepdims=True)
        out_ref[...] = l3sel + l1x_out + psqt

    full = lambda i: (0, 0)
    return pl.pallas_call(
        body,
        grid=(nblk,),
        in_specs=[
            pl.BlockSpec((blk, DP), lambda i: (i, 0)),
            pl.BlockSpec((blk, DP), lambda i: (i + nblk, 0)),
            pl.BlockSpec((blk, 1), lambda i: (i, 0)),
            pl.BlockSpec((blk, 1), lambda i: (i, 0)),
            pl.BlockSpec((blk, 1), lambda i: (i, 0)),
            pl.BlockSpec((1, L1), full),
            pl.BlockSpec((L1, NUM_LS * (L2 + 1)), full),
            pl.BlockSpec((1, NUM_LS * (L2 + 1)), full),
            pl.BlockSpec((2 * L2, NUM_LS * L3), full),
            pl.BlockSpec((1, NUM_LS * L3), full),
            pl.BlockSpec((L3, NUM_LS), full),
            pl.BlockSpec((1, NUM_LS), full),
        ],
        out_specs=pl.BlockSpec((blk, 1), lambda i: (i, 0)),
        out_shape=jax.ShapeDtypeStruct((batch, 1), jnp.float32),
    )(bags, bags, us, pidx, lsidx, fb_main, w1t, b1r, w2t, b2r, w3t, b3r)


def kernel(us, white_indices, black_indices, psqt_indices,
           layer_stack_indices, feat_W, feat_b, W1, b1, W2, b2, W3, b3):
    idx = jnp.concatenate([white_indices, black_indices], axis=0)
    idx = idx.astype(jnp.int32)
    tbl16 = _prep_table_tc(feat_W)
    bags = _embedding_bag_sc(tbl16, idx)

    fb_main = feat_b[:L1].reshape(1, L1)
    w1t = W1.reshape(NUM_LS * (L2 + 1), L1).T
    b1r = b1.reshape(1, NUM_LS * (L2 + 1))
    w2t = W2.reshape(NUM_LS * L3, 2 * L2).T
    b2r = b2.reshape(1, NUM_LS * L3)
    w3t = W3.reshape(NUM_LS, L3).T
    b3r = b3.reshape(1, NUM_LS)
    pidx = psqt_indices.reshape(-1, 1).astype(jnp.int32)
    lsidx = layer_stack_indices.reshape(-1, 1).astype(jnp.int32)
    return _dense_tc(bags, us.astype(jnp.float32), pidx, lsidx,
                     fb_main, w1t, b1r, w2t, b2r, w3t, b3r)


# speedup vs baseline: 1.3914x; 1.3914x over previous
import functools

import jax
import jax.numpy as jnp
from jax import lax
from jax.experimental import pallas as pl
from jax.experimental.pallas import tpu as pltpu
from jax.experimental.pallas import tpu_sc as plsc

L1 = 512
L2 = 16
L3 = 32
NUM_PSQT = 8
NUM_LS = 8
D = L1 + NUM_PSQT
DT = 544
DP = DT
KA = 32
NC = 2
NS = 16
NW = NC * NS


def _embedding_bag_sc(table16, idx):
    nbags = idx.shape[0]
    bags_per_w = nbags // NW
    mesh = plsc.VectorSubcoreMesh(
        core_axis_name="c", subcore_axis_name="s",
        num_cores=NC, num_subcores=NS)

    nbuf = 4
    nchunk = DT // 32

    @functools.partial(
        pl.kernel,
        out_type=jax.ShapeDtypeStruct((nbags, DP), jnp.float32),
        mesh=mesh,
        compiler_params=pltpu.CompilerParams(use_tc_tiling_on_sc=False,
                                             needs_layout_passes=False),
        scratch_types=[
            pltpu.VMEM((bags_per_w, KA), jnp.int32),
            [pltpu.VMEM((KA, DT // 2), jnp.int32) for _ in range(nbuf)],
            [pltpu.VMEM((DP,), jnp.float32) for _ in range(2)],
            [pltpu.SemaphoreType.DMA for _ in range(nbuf)],
            [pltpu.SemaphoreType.DMA for _ in range(2)],
        ],
    )
    def k(table_hbm, idx_hbm, out_hbm, idx_v, bufs, stages, sems, osems):
        wid = lax.axis_index("s") * NC + lax.axis_index("c")
        base = wid * bags_per_w
        pltpu.sync_copy(idx_hbm.at[pl.ds(base, bags_per_w)], idx_v)

        zero16 = jnp.zeros((16,), jnp.float32)

        for b in range(nbuf - 1):
            pltpu.async_copy(table_hbm.at[idx_v.at[b]],
                             bufs[b], sems[b])

        def group_body(p, carry):
            for b in range(nbuf):
                g = p * nbuf + b
                par = b % 2
                stage, osem = stages[par], osems[par]
                gn = g + nbuf - 1
                nb = (b + nbuf - 1) % nbuf

                @pl.when(gn < bags_per_w)
                def _issue():
                    pltpu.async_copy(table_hbm.at[idx_v.at[gn]],
                                     bufs[nb], sems[nb])

                pltpu.make_async_copy(
                    table_hbm.at[idx_v.at[g]], bufs[b], sems[b]).wait()

                @pl.when(g >= 2)
                def _wait_out():
                    pltpu.make_async_copy(
                        stage, out_hbm.at[base], osem).wait()

                def row_body(j, accs):
                    new = []
                    for c in range(nchunk):
                        v = bufs[b][j, pl.ds(c * 16, 16)]
                        lo = plsc.bitcast(v << 16, jnp.float32)
                        hi = plsc.bitcast(v & jnp.int32(-65536), jnp.float32)
                        new.append(accs[2 * c] + lo)
                        new.append(accs[2 * c + 1] + hi)
                    return tuple(new)

                accs = lax.fori_loop(0, KA, row_body,
                                     tuple(zero16 for _ in range(2 * nchunk)))
                for c in range(nchunk):
                    stage[pl.ds(16 * c, 16)] = accs[2 * c]
                    stage[pl.ds(DT // 2 + 16 * c, 16)] = accs[2 * c + 1]
                pltpu.async_copy(stage, out_hbm.at[base + g], osem)
            return carry

        lax.fori_loop(0, bags_per_w // nbuf, group_body, 0)
        for par in range(2):
            pltpu.make_async_copy(stages[par], out_hbm.at[base],
                                  osems[par]).wait()

    return k(table16, idx)


def _dense_tc(bags, us, pidx, lsidx, fb_main, w1t, b1r, w2t, b2r, w3t, b3r):
    batch = us.shape[0]
    blk = 512
    nblk = batch // blk
    scale = 127.0 / 128.0

    def body(wp_ref, bp_ref, us_ref, pidx_ref, ls_ref, fb_ref,
             w1_ref, b1_ref, w2_ref, b2_ref, w3_ref, b3_ref, out_ref):
        wp = wp_ref[...]
        bp = bp_ref[...]
        usv = us_ref[...]
        fb = fb_ref[...]
        w = jnp.clip(wp[:, :L1] + fb, 0.0, 1.0)
        b = jnp.clip(bp[:, :L1] + fb, 0.0, 1.0)
        half = L1 // 2
        w_pw = w[:, :half] * w[:, half:L1]
        b_pw = b[:, :half] * b[:, half:L1]
        usb = usv > 0.5
        l0 = jnp.concatenate(
            [jnp.where(usb, w_pw, b_pw), jnp.where(usb, b_pw, w_pw)],
            axis=1) * scale

        pidxv = pidx_ref[...]
        poh = (pidxv == lax.broadcasted_iota(jnp.int32, (blk, NUM_PSQT), 1)
               ).astype(jnp.float32)
        dps = wp[:, L1:D] - bp[:, L1:D]
        psqt = jnp.sum(dps * poh, axis=1, keepdims=True) * (usv - 0.5)

        lsv = ls_ref[...]
        l1x_all = jnp.dot(l0, w1_ref[...],
                          preferred_element_type=jnp.float32) + b1_ref[...]
        nsel = L2 + 1
        sel1 = jnp.zeros((blk, nsel), jnp.float32)
        for kk in range(NUM_LS):
            sel1 = sel1 + jnp.where(lsv == kk,
                                    l1x_all[:, kk * nsel:(kk + 1) * nsel], 0.0)
        l1x_ = sel1[:, :L2]
        l1x_out = sel1[:, L2:L2 + 1]
        l1c = jnp.clip(l1x_, 0.0, 1.0)
        l1cat = jnp.concatenate([l1c * l1c, l1c], axis=1) * scale

        l2x_all = jnp.dot(l1cat, w2_ref[...],
                          preferred_element_type=jnp.float32) + b2_ref[...]
        sel2 = jnp.zeros((blk, L3), jnp.float32)
        for kk in range(NUM_LS):
            sel2 = sel2 + jnp.where(lsv == kk,
                                    l2x_all[:, kk * L3:(kk + 1) * L3], 0.0)
        l2x = jnp.clip(sel2, 0.0, 1.0)

        l3_all = jnp.dot(l2x, w3_ref[...],
                         preferred_element_type=jnp.float32) + b3_ref[...]
        lsoh = (lsv == lax.broadcasted_iota(jnp.int32, (blk, NUM_LS), 1)
                ).astype(jnp.float32)
        l3sel = jnp.sum(l3_all * lsoh, axis=1, keepdims=True)
        out_ref[...] = l3sel + l1x_out + psqt

    full = lambda i: (0, 0)
    return pl.pallas_call(
        body,
        grid=(nblk,),
        in_specs=[
            pl.BlockSpec((blk, DP), lambda i: (i, 0)),
            pl.BlockSpec((blk, DP), lambda i: (i + nblk, 0)),
            pl.BlockSpec((blk, 1), lambda i: (i, 0)),
            pl.BlockSpec((blk, 1), lambda i: (i, 0)),
            pl.BlockSpec((blk, 1), lambda i: (i, 0)),
            pl.BlockSpec((1, L1), full),
            pl.BlockSpec((L1, NUM_LS * (L2 + 1)), full),
            pl.BlockSpec((1, NUM_LS * (L2 + 1)), full),
            pl.BlockSpec((2 * L2, NUM_LS * L3), full),
            pl.BlockSpec((1, NUM_LS * L3), full),
            pl.BlockSpec((L3, NUM_LS), full),
            pl.BlockSpec((1, NUM_LS), full),
        ],
        out_specs=pl.BlockSpec((blk, 1), lambda i: (i, 0)),
        out_shape=jax.ShapeDtypeStruct((batch, 1), jnp.float32),
    )(bags, bags, us, pidx, lsidx, fb_main, w1t, b1r, w2t, b2r, w3t, b3r)


def kernel(us, white_indices, black_indices, psqt_indices,
           layer_stack_indices, feat_W, feat_b, W1, b1, W2, b2, W3, b3):
    idx = jnp.concatenate([white_indices, black_indices], axis=0)
    idx = idx.astype(jnp.int32)
    half = DT // 2
    xb = feat_W.astype(jnp.bfloat16)
    lo = jax.lax.bitcast_convert_type(xb[:, :half], jnp.uint16)
    hi = jax.lax.bitcast_convert_type(xb[:, half:], jnp.uint16)
    lo32 = lo.astype(jnp.uint32)
    hi32 = jnp.pad(hi.astype(jnp.uint32), ((0, 0), (0, DT - D)))
    tblp = jax.lax.bitcast_convert_type(lo32 | (hi32 << 16), jnp.int32)
    bags = _embedding_bag_sc(tblp, idx)

    fb_main = feat_b[:L1].reshape(1, L1)
    w1t = W1.reshape(NUM_LS * (L2 + 1), L1).T
    b1r = b1.reshape(1, NUM_LS * (L2 + 1))
    w2t = W2.reshape(NUM_LS * L3, 2 * L2).T
    b2r = b2.reshape(1, NUM_LS * L3)
    w3t = W3.reshape(NUM_LS, L3).T
    b3r = b3.reshape(1, NUM_LS)
    pidx = psqt_indices.reshape(-1, 1).astype(jnp.int32)
    lsidx = layer_stack_indices.reshape(-1, 1).astype(jnp.int32)
    return _dense_tc(bags, us.astype(jnp.float32), pidx, lsidx,
                     fb_main, w1t, b1r, w2t, b2r, w3t, b3r)
